# baseline (speedup 1.0000x reference)
import functools

import jax
import jax.numpy as jnp
from jax import lax
from jax.experimental import pallas as pl
from jax.experimental.pallas import tpu as pltpu
from jax.experimental.pallas import tpu_sc as plsc

B, N2, N1, C, O, K = 4, 8192, 2048, 256, 256, 3
TILE = 512
KP = 4
NW = 32
ROWS_W = (B * N2) // NW
CH = 32
NCH = ROWS_W // CH


def _top3_body(xdn2_ref, xs_ref, idx_ref, w_ref):
    bidx = pl.program_id(0)
    xdn2 = xdn2_ref[0]
    xs = xs_ref[0]
    x2 = 0.25 * jnp.sum(xdn2 * xdn2, axis=1, keepdims=True)
    y2 = jnp.sum(xs * xs, axis=1)[None, :]
    xy_n2 = jax.lax.dot_general(
        xdn2, xs, (((1,), (1,)), ((), ())), preferred_element_type=jnp.float32
    )
    dist2 = jnp.maximum((x2 + y2) + xy_n2, 1e-12)

    big = jnp.float32(jnp.inf)
    iotaf = jax.lax.broadcasted_iota(jnp.int32, dist2.shape, 1).astype(jnp.float32)
    nf = jnp.float32(N1)
    d = dist2
    ams = []
    ws = []
    wsum = jnp.zeros((dist2.shape[0], 1), jnp.float32)
    for k in range(K):
        m = jnp.min(d, axis=1, keepdims=True)
        t = jnp.where(d == m, iotaf, nf)
        am = jnp.min(t, axis=1, keepdims=True)
        w = jax.lax.rsqrt(m)
        ams.append(am)
        ws.append(w)
        wsum = wsum + w
        if k < K - 1:
            d = jnp.where(t == am, big, d)

    zero_i = jnp.zeros(ams[0].shape, jnp.int32)
    gbase = bidx * N1
    idx_ref[0] = jnp.concatenate(
        [a.astype(jnp.int32) + gbase for a in ams] + [zero_i], axis=1
    )
    zero_f = jnp.zeros(ws[0].shape, jnp.float32)
    w_ref[0] = jnp.concatenate([w / wsum for w in ws] + [zero_f], axis=1)


@jax.jit
def _top3(xdn2, xyz_sparse):
    grid = (B, N2 // TILE)
    return pl.pallas_call(
        _top3_body,
        grid=grid,
        in_specs=[
            pl.BlockSpec((1, TILE, 3), lambda b, t: (b, t, 0)),
            pl.BlockSpec((1, N1, 3), lambda b, t: (b, 0, 0)),
        ],
        out_specs=[
            pl.BlockSpec((1, TILE, KP), lambda b, t: (b, t, 0)),
            pl.BlockSpec((1, TILE, KP), lambda b, t: (b, t, 0)),
        ],
        out_shape=[
            jax.ShapeDtypeStruct((B, N2, KP), jnp.int32),
            jax.ShapeDtypeStruct((B, N2, KP), jnp.float32),
        ],
    )(xdn2, xyz_sparse)


def _sc_gather_body(idx_hbm, w_hbm, feat_hbm, out_hbm, idx_v, w_v, rows_v,
                    out_v, sem):
    wid = lax.axis_index("s") * 2 + lax.axis_index("c")
    base_row = wid * ROWS_W
    pltpu.sync_copy(idx_hbm.at[pl.ds(base_row * KP, ROWS_W * KP)], idx_v)
    pltpu.sync_copy(w_hbm.at[pl.ds(base_row * KP, ROWS_W * KP)], w_v.at[pl.ds(0, ROWS_W * KP)])

    def chunk_body(c):
        idx_slice = idx_v.at[pl.ds(c * CH * KP, CH * KP)]
        pltpu.async_copy(feat_hbm.at[idx_slice], rows_v, sem).wait()

        def row_body(i):
            wrow = w_v[pl.ds((c * CH + i) * KP, 16)]
            for j in range(C // 16):
                acc = jnp.zeros((16,), jnp.float32)
                for k in range(KP - 1):
                    acc = acc + wrow[k] * rows_v[i * KP + k, pl.ds(j * 16, 16)]
                out_v[i, pl.ds(j * 16, 16)] = acc

        pl.loop(0, CH)(row_body)
        pltpu.sync_copy(out_v, out_hbm.at[pl.ds(base_row + c * CH, CH)])

    pl.loop(0, NCH)(chunk_body)


@jax.jit
def _sc_gather(idx4, w4, feat_flat):
    mesh = plsc.VectorSubcoreMesh(core_axis_name="c", subcore_axis_name="s")
    kfn = functools.partial(
        pl.kernel,
        mesh=mesh,
        out_type=jax.ShapeDtypeStruct((B * N2, C), jnp.float32),
        scratch_types=[
            pltpu.VMEM((ROWS_W * KP,), jnp.int32),
            pltpu.VMEM((ROWS_W * KP + 16,), jnp.float32),
            pltpu.VMEM((CH * KP, C), jnp.float32),
            pltpu.VMEM((CH, C), jnp.float32),
            pltpu.SemaphoreType.DMA,
        ],
    )(_sc_gather_body)
    return kfn(idx4.reshape(-1), w4.reshape(-1), feat_flat)


def _mlp_body(x_ref, w1_ref, b1_ref, w2_ref, b2_ref, out_ref):
    h = jnp.maximum(
        jax.lax.dot_general(
            x_ref[...].astype(jnp.bfloat16), w1_ref[...],
            (((1,), (0,)), ((), ())), preferred_element_type=jnp.float32,
        ) + b1_ref[...],
        0.0,
    )
    out_ref[...] = jax.lax.dot_general(
        h.astype(jnp.bfloat16), w2_ref[...],
        (((1,), (0,)), ((), ())), preferred_element_type=jnp.float32,
    ) + b2_ref[...]


@jax.jit
def _mlp(x, w1_bf, b1r, w2_bf, b2r):
    tile = 2048
    return pl.pallas_call(
        _mlp_body,
        grid=(B * N2 // tile,),
        in_specs=[
            pl.BlockSpec((tile, C), lambda t: (t, 0)),
            pl.BlockSpec((C, O), lambda t: (0, 0)),
            pl.BlockSpec((1, O), lambda t: (0, 0)),
            pl.BlockSpec((O, O), lambda t: (0, 0)),
            pl.BlockSpec((1, O), lambda t: (0, 0)),
        ],
        out_specs=pl.BlockSpec((tile, O), lambda t: (t, 0)),
        out_shape=jax.ShapeDtypeStruct((B * N2, O), jnp.float32),
    )(x, w1_bf, b1r, w2_bf, b2r)


@jax.jit
def kernel(xyz_dense, xyz_sparse, feat_sparse, W1, b1, W2, b2):
    xdn2 = xyz_dense * jnp.float32(-2.0)
    idx4, w4 = _top3(xdn2, xyz_sparse)
    feat_flat = feat_sparse.reshape(B * N1, C)
    interp = _sc_gather(idx4, w4, feat_flat)
    out = _mlp(interp, W1.astype(jnp.bfloat16), b1.reshape(1, O),
               W2.astype(jnp.bfloat16), b2.reshape(1, O))
    return out.reshape(B, N2, O)


# speedup vs baseline: 4.6460x; 4.6460x over previous
import jax
import jax.numpy as jnp
from jax.experimental import pallas as pl

B, N2, N1, C, O, K = 4, 8192, 2048, 256, 256, 3
TILE = 512


def _fp_body(xdn2_ref, xs_ref, fs_ref, w1_ref, b1_ref, w2_ref, b2_ref, out_ref):
    xdn2 = xdn2_ref[0]
    xs = xs_ref[0]
    x2 = 0.25 * jnp.sum(xdn2 * xdn2, axis=1, keepdims=True)
    y2 = jnp.sum(xs * xs, axis=1)[None, :]
    xy_n2 = jax.lax.dot_general(
        xdn2, xs, (((1,), (1,)), ((), ())), preferred_element_type=jnp.float32
    )
    dist2 = jnp.maximum((x2 + y2) + xy_n2, 1e-12)

    big = jnp.float32(jnp.inf)
    iotaf = jax.lax.broadcasted_iota(jnp.int32, (1, N1), 1).astype(jnp.float32)
    nf = jnp.float32(N1)
    acc = jnp.zeros(dist2.shape, jnp.float32)
    wsum = jnp.zeros((dist2.shape[0], 1), jnp.float32)
    d = dist2
    for k in range(K):
        m = jnp.min(d, axis=1, keepdims=True)
        t = jnp.where(d == m, iotaf, nf)
        am = jnp.min(t, axis=1, keepdims=True)
        sel = t == am
        w = jax.lax.rsqrt(m)
        acc = acc + jnp.where(sel, w, 0.0)
        wsum = wsum + w
        if k < K - 1:
            d = jnp.where(sel, big, d)

    feat = jax.lax.dot_general(
        acc.astype(jnp.bfloat16), fs_ref[0], (((1,), (0,)), ((), ())), preferred_element_type=jnp.float32
    ) / wsum
    h = jnp.maximum(
        jax.lax.dot_general(
            feat.astype(jnp.bfloat16), w1_ref[...],
            (((1,), (0,)), ((), ())), preferred_element_type=jnp.float32,
        ) + b1_ref[...],
        0.0,
    )
    out_ref[0] = jax.lax.dot_general(
        h.astype(jnp.bfloat16), w2_ref[...],
        (((1,), (0,)), ((), ())), preferred_element_type=jnp.float32,
    ) + b2_ref[...]


@jax.jit
def kernel(xyz_dense, xyz_sparse, feat_sparse, W1, b1, W2, b2):
    xdn2 = xyz_dense * jnp.float32(-2.0)
    fs_bf = feat_sparse.astype(jnp.bfloat16)
    w1_bf = W1.astype(jnp.bfloat16)
    w2_bf = W2.astype(jnp.bfloat16)
    b1r = b1.reshape(1, O)
    b2r = b2.reshape(1, O)
    grid = (B, N2 // TILE)
    return pl.pallas_call(
        _fp_body,
        grid=grid,
        in_specs=[
            pl.BlockSpec((1, TILE, 3), lambda b, t: (b, t, 0)),
            pl.BlockSpec((1, N1, 3), lambda b, t: (b, 0, 0)),
            pl.BlockSpec((1, N1, C), lambda b, t: (b, 0, 0)),
            pl.BlockSpec((C, O), lambda b, t: (0, 0)),
            pl.BlockSpec((1, O), lambda b, t: (0, 0)),
            pl.BlockSpec((O, O), lambda b, t: (0, 0)),
            pl.BlockSpec((1, O), lambda b, t: (0, 0)),
        ],
        out_specs=pl.BlockSpec((1, TILE, O), lambda b, t: (b, t, 0)),
        out_shape=jax.ShapeDtypeStruct((B, N2, O), jnp.float32),
    )(xdn2, xyz_sparse, fs_bf, w1_bf, b1r, w2_bf, b2r)


# speedup vs baseline: 4.8909x; 1.0527x over previous
import jax
import jax.numpy as jnp
from jax.experimental import pallas as pl
from jax.experimental.pallas import tpu as pltpu

B, N2, N1, C, O, K = 4, 8192, 2048, 256, 256, 3
TILE = 512


def _fp_body(xd_ref, xs_ref, fs_ref, w1_ref, b1_ref, w2_ref, b2_ref, out_ref,
             fsb_ref):
    @pl.when(pl.program_id(1) == 0)
    def _():
        fsb_ref[...] = fs_ref[0].astype(jnp.bfloat16)

    xd = xd_ref[0]
    xs = xs_ref[0]
    xdn2 = xd * jnp.float32(-2.0)
    x2 = jnp.sum(xd * xd, axis=1, keepdims=True)
    y2 = jnp.sum(xs * xs, axis=1)[None, :]
    xy_n2 = jax.lax.dot_general(
        xdn2, xs, (((1,), (1,)), ((), ())), preferred_element_type=jnp.float32
    )
    dist2 = jnp.maximum((x2 + y2) + xy_n2, 1e-12)

    big = jnp.float32(jnp.inf)
    iotaf = jax.lax.broadcasted_iota(jnp.int32, (1, N1), 1).astype(jnp.float32)
    nf = jnp.float32(N1)
    acc = jnp.zeros(dist2.shape, jnp.float32)
    wsum = jnp.zeros((dist2.shape[0], 1), jnp.float32)
    d = dist2
    for k in range(K):
        m = jnp.min(d, axis=1, keepdims=True)
        t = jnp.where(d == m, iotaf, nf)
        am = jnp.min(t, axis=1, keepdims=True)
        sel = t == am
        w = jax.lax.rsqrt(m)
        acc = acc + jnp.where(sel, w, 0.0)
        wsum = wsum + w
        if k < K - 1:
            d = jnp.where(sel, big, d)

    feat = jax.lax.dot_general(
        acc.astype(jnp.bfloat16), fsb_ref[...], (((1,), (0,)), ((), ())),
        preferred_element_type=jnp.float32,
    ) / wsum
    h = jnp.maximum(
        jax.lax.dot_general(
            feat.astype(jnp.bfloat16), w1_ref[...].astype(jnp.bfloat16),
            (((1,), (0,)), ((), ())), preferred_element_type=jnp.float32,
        ) + b1_ref[...],
        0.0,
    )
    out_ref[0] = jax.lax.dot_general(
        h.astype(jnp.bfloat16), w2_ref[...].astype(jnp.bfloat16),
        (((1,), (0,)), ((), ())), preferred_element_type=jnp.float32,
    ) + b2_ref[...]


@jax.jit
def kernel(xyz_dense, xyz_sparse, feat_sparse, W1, b1, W2, b2):
    b1r = b1.reshape(1, O)
    b2r = b2.reshape(1, O)
    grid = (B, N2 // TILE)
    return pl.pallas_call(
        _fp_body,
        grid=grid,
        in_specs=[
            pl.BlockSpec((1, TILE, 3), lambda b, t: (b, t, 0)),
            pl.BlockSpec((1, N1, 3), lambda b, t: (b, 0, 0)),
            pl.BlockSpec((1, N1, C), lambda b, t: (b, 0, 0)),
            pl.BlockSpec((C, O), lambda b, t: (0, 0)),
            pl.BlockSpec((1, O), lambda b, t: (0, 0)),
            pl.BlockSpec((O, O), lambda b, t: (0, 0)),
            pl.BlockSpec((1, O), lambda b, t: (0, 0)),
        ],
        out_specs=pl.BlockSpec((1, TILE, O), lambda b, t: (b, t, 0)),
        out_shape=jax.ShapeDtypeStruct((B, N2, O), jnp.float32),
        scratch_shapes=[pltpu.VMEM((N1, C), jnp.bfloat16)],
    )(xyz_dense, xyz_sparse, feat_sparse, W1, b1r, W2, b2r)


# speedup vs baseline: 4.9734x; 1.0169x over previous
import jax
import jax.numpy as jnp
from jax.experimental import pallas as pl
from jax.experimental.pallas import tpu as pltpu

B, N2, N1, C, O, K = 4, 8192, 2048, 256, 256, 3
TILE = 512


def _fp_body(xd_ref, xs_ref, fs_ref, w1_ref, b1_ref, w2_ref, b2_ref, out_ref,
             fsb_ref):
    @pl.when(pl.program_id(1) == 0)
    def _():
        fsb_ref[...] = fs_ref[0].astype(jnp.bfloat16)

    xd = xd_ref[0]
    xs = xs_ref[0]
    xdn2 = xd * jnp.float32(-2.0)
    x2 = jnp.sum(xd * xd, axis=1, keepdims=True)
    y2 = jnp.sum(xs * xs, axis=1)[None, :]
    xy_n2 = jax.lax.dot_general(
        xdn2, xs, (((1,), (1,)), ((), ())), preferred_element_type=jnp.float32
    )
    dist2 = jnp.maximum((x2 + y2) + xy_n2, 1e-12)

    big = jnp.float32(jnp.inf)
    iotaf = jax.lax.broadcasted_iota(jnp.int32, (1, N1), 1).astype(jnp.float32)
    nf = jnp.float32(N1)
    wsum = jnp.zeros((dist2.shape[0], 1), jnp.float32)
    d = dist2
    sels, ws = [], []
    for k in range(K):
        m = jnp.min(d, axis=1, keepdims=True)
        t = jnp.where(d == m, iotaf, nf)
        am = jnp.min(t, axis=1, keepdims=True)
        sel = t == am
        w = jax.lax.rsqrt(m)
        sels.append(sel)
        ws.append(w)
        wsum = wsum + w
        if k < K - 1:
            d = jnp.where(sel, big, d)

    winv = 1.0 / wsum
    acc = jnp.zeros(dist2.shape, jnp.float32)
    for sel, w in zip(reversed(sels), reversed(ws)):
        acc = jnp.where(sel, w * winv, acc)

    feat = jax.lax.dot_general(
        acc.astype(jnp.bfloat16), fsb_ref[...], (((1,), (0,)), ((), ())),
        preferred_element_type=jnp.float32,
    )
    h = jnp.maximum(
        jax.lax.dot_general(
            feat.astype(jnp.bfloat16), w1_ref[...].astype(jnp.bfloat16),
            (((1,), (0,)), ((), ())), preferred_element_type=jnp.float32,
        ) + b1_ref[...],
        0.0,
    )
    out_ref[0] = jax.lax.dot_general(
        h.astype(jnp.bfloat16), w2_ref[...].astype(jnp.bfloat16),
        (((1,), (0,)), ((), ())), preferred_element_type=jnp.float32,
    ) + b2_ref[...]


@jax.jit
def kernel(xyz_dense, xyz_sparse, feat_sparse, W1, b1, W2, b2):
    b1r = b1.reshape(1, O)
    b2r = b2.reshape(1, O)
    grid = (B, N2 // TILE)
    return pl.pallas_call(
        _fp_body,
        grid=grid,
        in_specs=[
            pl.BlockSpec((1, TILE, 3), lambda b, t: (b, t, 0)),
            pl.BlockSpec((1, N1, 3), lambda b, t: (b, 0, 0)),
            pl.BlockSpec((1, N1, C), lambda b, t: (b, 0, 0)),
            pl.BlockSpec((C, O), lambda b, t: (0, 0)),
            pl.BlockSpec((1, O), lambda b, t: (0, 0)),
            pl.BlockSpec((O, O), lambda b, t: (0, 0)),
            pl.BlockSpec((1, O), lambda b, t: (0, 0)),
        ],
        out_specs=pl.BlockSpec((1, TILE, O), lambda b, t: (b, t, 0)),
        out_shape=jax.ShapeDtypeStruct((B, N2, O), jnp.float32),
        scratch_shapes=[pltpu.VMEM((N1, C), jnp.bfloat16)],
    )(xyz_dense, xyz_sparse, feat_sparse, W1, b1r, W2, b2r)


# speedup vs baseline: 4.9867x; 1.0027x over previous
import jax
import jax.numpy as jnp
from jax.experimental import pallas as pl
from jax.experimental.pallas import tpu as pltpu

B, N2, N1, C, O, K = 4, 8192, 2048, 256, 256, 3
TILE = 512


def _fp_body(xd_ref, xs_ref, fs_ref, w1_ref, b1_ref, w2_ref, b2_ref, out_ref):
    xd = xd_ref[0]
    xs = xs_ref[0]
    xdn2 = xd * jnp.float32(-2.0)
    x2 = jnp.sum(xd * xd, axis=1, keepdims=True)
    y2 = jnp.sum(xs * xs, axis=1)[None, :]
    xy_n2 = jax.lax.dot_general(
        xdn2, xs, (((1,), (1,)), ((), ())), preferred_element_type=jnp.float32
    )
    dist2 = jnp.maximum((x2 + y2) + xy_n2, 1e-12)

    big = jnp.float32(jnp.inf)
    iotaf = jax.lax.broadcasted_iota(jnp.int32, (1, N1), 1).astype(jnp.float32)
    nf = jnp.float32(N1)
    wsum = jnp.zeros((dist2.shape[0], 1), jnp.float32)
    d = dist2
    sels, ws = [], []
    for k in range(K):
        m = jnp.min(d, axis=1, keepdims=True)
        t = jnp.where(d == m, iotaf, nf)
        am = jnp.min(t, axis=1, keepdims=True)
        sel = t == am
        w = jax.lax.rsqrt(m)
        sels.append(sel)
        ws.append(w)
        wsum = wsum + w
        if k < K - 1:
            d = jnp.where(sel, big, d)

    winv = 1.0 / wsum
    acc = jnp.zeros(dist2.shape, jnp.float32)
    for sel, w in zip(reversed(sels), reversed(ws)):
        acc = jnp.where(sel, w * winv, acc)

    feat = jax.lax.dot_general(
        acc, fs_ref[0], (((1,), (0,)), ((), ())),
        preferred_element_type=jnp.float32,
    )
    h = jnp.maximum(
        jax.lax.dot_general(
            feat.astype(jnp.bfloat16), w1_ref[...].astype(jnp.bfloat16),
            (((1,), (0,)), ((), ())), preferred_element_type=jnp.float32,
        ) + b1_ref[...],
        0.0,
    )
    out_ref[0] = jax.lax.dot_general(
        h.astype(jnp.bfloat16), w2_ref[...].astype(jnp.bfloat16),
        (((1,), (0,)), ((), ())), preferred_element_type=jnp.float32,
    ) + b2_ref[...]


@jax.jit
def kernel(xyz_dense, xyz_sparse, feat_sparse, W1, b1, W2, b2):
    b1r = b1.reshape(1, O)
    b2r = b2.reshape(1, O)
    grid = (B, N2 // TILE)
    return pl.pallas_call(
        _fp_body,
        grid=grid,
        in_specs=[
            pl.BlockSpec((1, TILE, 3), lambda b, t: (b, t, 0)),
            pl.BlockSpec((1, N1, 3), lambda b, t: (b, 0, 0)),
            pl.BlockSpec((1, N1, C), lambda b, t: (b, 0, 0)),
            pl.BlockSpec((C, O), lambda b, t: (0, 0)),
            pl.BlockSpec((1, O), lambda b, t: (0, 0)),
            pl.BlockSpec((O, O), lambda b, t: (0, 0)),
            pl.BlockSpec((1, O), lambda b, t: (0, 0)),
        ],
        out_specs=pl.BlockSpec((1, TILE, O), lambda b, t: (b, t, 0)),
        out_shape=jax.ShapeDtypeStruct((B, N2, O), jnp.float32),
    )(xyz_dense, xyz_sparse, feat_sparse, W1, b1r, W2, b2r)
